# Initial kernel scaffold; baseline (speedup 1.0000x reference)
import jax
import jax.numpy as jnp
from jax.experimental import pallas as pl


def kernel(h, params, edge_index, node_graph_ids, node_role_id, drug_pairs):
    raise NotImplementedError("write your pallas kernel here")


# speedup vs baseline: 21.0399x; 21.0399x over previous
import functools

import jax
import jax.numpy as jnp
from jax import lax
from jax.experimental import pallas as pl
from jax.experimental.pallas import tpu as pltpu
from jax.experimental.pallas import tpu_sc as plsc

_NH = 8


def _dot_t(x, w):
    return lax.dot_general(x, w, (((1,), (1,)), ((), ())),
                           preferred_element_type=jnp.float32)


def _gelu(x):
    return 0.5 * x * (1.0 + lax.erf(x * jnp.float32(0.7071067811865476)))


def _qkv(h, wq, bq, wk, bk, wv, bv, scale):
    n, d = h.shape

    def body(h_ref, wq_ref, bq_ref, wk_ref, bk_ref, wv_ref, bv_ref,
             q_ref, k_ref, v_ref):
        hh = h_ref[...]
        q_ref[...] = (_dot_t(hh, wq_ref[...]) + bq_ref[...]) * scale
        k_ref[...] = _dot_t(hh, wk_ref[...]) + bk_ref[...]
        v_ref[...] = _dot_t(hh, wv_ref[...]) + bv_ref[...]

    out = jax.ShapeDtypeStruct((n, d), jnp.float32)
    return pl.pallas_call(body, out_shape=[out, out, out])(
        h, wq, bq, wk, bk, wv, bv)


def _post1(agg, h, wo, bo, g1, b1):
    n, d = h.shape

    def body(a_ref, h_ref, wo_ref, bo_ref, g_ref, b_ref, y_ref):
        x = _dot_t(a_ref[...], wo_ref[...]) + bo_ref[...] + h_ref[...]
        mu = jnp.mean(x, axis=0, keepdims=True)
        var = jnp.mean((x - mu) ** 2, axis=0, keepdims=True)
        xn = (x - mu) / jnp.sqrt(var + 1e-5) * g_ref[...] + b_ref[...]
        y_ref[...] = _gelu(xn)

    return pl.pallas_call(body, out_shape=jax.ShapeDtypeStruct((n, d), jnp.float32))(
        agg, h, wo, bo, g1, b1)


def _post2(y, w1, c1, w2, c2, g2, b2):
    n, d = y.shape

    def body(y_ref, w1_ref, c1_ref, w2_ref, c2_ref, g_ref, b_ref, o_ref):
        yy = y_ref[...]
        t = _gelu(_dot_t(yy, w1_ref[...]) + c1_ref[...])
        f = _dot_t(t, w2_ref[...]) + c2_ref[...]
        z = yy + f
        mu = jnp.mean(z, axis=0, keepdims=True)
        var = jnp.mean((z - mu) ** 2, axis=0, keepdims=True)
        o_ref[...] = (z - mu) / jnp.sqrt(var + 1e-5) * g_ref[...] + b_ref[...]

    return pl.pallas_call(body, out_shape=jax.ShapeDtypeStruct((n, d), jnp.float32))(
        y, w1, c1, w2, c2, g2, b2)


def _readout(h, wh, wb, centers, labels, npg):
    n, d = h.shape
    b = n // npg
    nd = centers.shape[0]
    nlab = labels.shape[0]

    def body(h_ref, wh_ref, wb_ref, c_ref, lab_ref, g_ref, loss_ref):
        hh = h_ref[...]
        hb = hh.reshape(b, npg, d)
        xh = hb[:, 0, :]
        xt = hb[:, 1, :]
        base_vec = jnp.concatenate([xh, xt], axis=1)
        tb = _dot_t(base_vec, wb_ref[...])
        th = _dot_t(hh, wh_ref[...])
        a = jnp.sum(th.reshape(b, npg, d) * tb[:, None, :], axis=2)
        m = jnp.max(a, axis=1, keepdims=True)
        e = jnp.exp(a - m)
        sm = e / jnp.sum(e, axis=1, keepdims=True)
        g_ref[...] = jnp.sum(hb * sm[:, :, None], axis=1) / float(npg)
        x32 = jnp.concatenate([xh, xt], axis=0)
        onehot = (lab_ref[...] ==
                  lax.broadcasted_iota(jnp.int32, (nlab, nd), 1)
                  ).astype(jnp.float32)
        cg = lax.dot_general(onehot, c_ref[...], (((1,), (0,)), ((), ())),
                             preferred_element_type=jnp.float32)
        diff = x32 - cg
        dist = jnp.sum(diff * diff, axis=1, keepdims=True)
        loss = jnp.sum(jnp.clip(dist, 1e-12, 1e12)) / float(nlab)
        loss_ref[...] = loss[None, None]

    return pl.pallas_call(body, out_shape=[
        jax.ShapeDtypeStruct((b, d), jnp.float32),
        jax.ShapeDtypeStruct((1, 1), jnp.float32),
    ])(h, wh, wb, centers, labels)


def _edge_attn(q, k, v, src_l, dst_l, nb, npg, epg, hd):
    n, d = q.shape
    nheads = d // hd
    hpt = nheads // 2
    npad = ((npg + 15) // 16) * 16
    mesh = plsc.VectorSubcoreMesh(core_axis_name="c", subcore_axis_name="s")

    @functools.partial(
        pl.kernel,
        out_type=jax.ShapeDtypeStruct((n, d), jnp.float32),
        mesh=mesh,
        compiler_params=pltpu.CompilerParams(use_tc_tiling_on_sc=False,
                                             needs_layout_passes=False),
        scratch_types=[
            pltpu.VMEM((npg, hd), jnp.float32),
            pltpu.VMEM((npg, hd), jnp.float32),
            pltpu.VMEM((npg, hd), jnp.float32),
            pltpu.VMEM((npg, hd), jnp.float32),
            pltpu.VMEM((npad,), jnp.float32),
            pltpu.VMEM((16,), jnp.float32),
            pltpu.VMEM((epg,), jnp.int32),
            pltpu.VMEM((epg,), jnp.int32),
            pltpu.VMEM((epg,), jnp.float32),
        ],
    )
    def ker(q_hbm, k_hbm, v_hbm, sl_hbm, dl_hbm, out_hbm,
            qt, kt, vt, ot, den, mx, sl, dl, sc):
        cid = lax.axis_index("c")
        sid = lax.axis_index("s")
        w = sid * 2 + cid
        b = w // 2
        half = w % 2
        ebase = b * epg
        nbase = b * npg
        pltpu.sync_copy(sl_hbm.at[pl.ds(ebase, epg)], sl)
        pltpu.sync_copy(dl_hbm.at[pl.ds(ebase, epg)], dl)

        @pl.loop(0, hpt)
        def _head(j):
            cb = (half * hpt + j) * hd
            pltpu.sync_copy(q_hbm.at[pl.ds(nbase, npg), pl.ds(cb, hd)], qt)
            pltpu.sync_copy(k_hbm.at[pl.ds(nbase, npg), pl.ds(cb, hd)], kt)
            pltpu.sync_copy(v_hbm.at[pl.ds(nbase, npg), pl.ds(cb, hd)], vt)

            @pl.loop(0, npad, step=16)
            def _zd(i):
                den[pl.ds(i, 16)] = jnp.zeros((16,), jnp.float32)

            @pl.loop(0, npg)
            def _zo(r):
                ot[r, :] = jnp.zeros((16,), jnp.float32)

            mx[...] = jnp.full((16,), -jnp.inf, jnp.float32)

            @pl.loop(0, epg, step=16)
            def _p1(i):
                s = sl[pl.ds(i, 16)]
                dv = dl[pl.ds(i, 16)]
                acc = jnp.zeros((16,), jnp.float32)
                for dd in range(hd):
                    col = jnp.full((16,), dd, jnp.int32)
                    acc = acc + (plsc.load_gather(qt, [s, col]) *
                                 plsc.load_gather(kt, [dv, col]))
                sc[pl.ds(i, 16)] = acc
                mx[...] = jnp.maximum(mx[...], acc)

            gm = jnp.max(mx[...])

            @pl.loop(0, epg, step=16)
            def _p2(i):
                e = jnp.exp(sc[pl.ds(i, 16)] - gm)
                sc[pl.ds(i, 16)] = e
                plsc.addupdate_scatter(den, [sl[pl.ds(i, 16)]], e)

            @pl.loop(0, epg, step=16)
            def _p3(i):
                s = sl[pl.ds(i, 16)]
                dv = dl[pl.ds(i, 16)]
                a = sc[pl.ds(i, 16)] / plsc.load_gather(den, [s])
                for dd in range(hd):
                    col = jnp.full((16,), dd, jnp.int32)
                    vv = plsc.load_gather(vt, [dv, col])
                    plsc.addupdate_scatter(ot, [s, col], a * vv)

            pltpu.sync_copy(ot, out_hbm.at[pl.ds(nbase, npg), pl.ds(cb, hd)])

    return ker(q, k, v, src_l, dst_l)


def kernel(h, params, edge_index, node_graph_ids, node_role_id, drug_pairs):
    n, d = h.shape
    b = drug_pairs.shape[0]
    npg = n // b
    e = edge_index.shape[1]
    epg = e // b
    hd = d // _NH
    scale = hd ** (-0.5)

    perm = jnp.arange(d).reshape(hd, _NH).T.reshape(-1)
    src_l = (edge_index[0] % npg).astype(jnp.int32)
    dst_l = (edge_index[1] % npg).astype(jnp.int32)

    hh = h
    for lp in params["layers"]:
        wq = lp["q"]["W"][perm, :]
        bq = lp["q"]["b"][perm][None, :]
        wk = lp["k"]["W"][perm, :]
        bk = lp["k"]["b"][perm][None, :]
        wv = lp["v"]["W"][perm, :]
        bv = lp["v"]["b"][perm][None, :]
        wo = lp["o"]["W"][:, perm]
        bo = lp["o"]["b"][None, :]
        q, k, v = _qkv(hh, wq, bq, wk, bk, wv, bv, scale)
        agg = _edge_attn(q, k, v, src_l, dst_l, b, npg, epg, hd)
        y = _post1(agg, hh, wo, bo,
                   lp["bn1"]["g"][None, :], lp["bn1"]["b"][None, :])
        hh = _post2(y, lp["ffn1"]["W"], lp["ffn1"]["b"][None, :],
                    lp["ffn2"]["W"], lp["ffn2"]["b"][None, :],
                    lp["bn2"]["g"][None, :], lp["bn2"]["b"][None, :])

    labels = jnp.concatenate([drug_pairs[:, 0], drug_pairs[:, 1]],
                             axis=0)[:, None].astype(jnp.int32)
    g_out, loss = _readout(hh, params["W_h"], params["W_base"],
                           params["centers"], labels, npg)
    return (hh, g_out, loss.reshape(()))
